# Initial kernel scaffold; baseline (speedup 1.0000x reference)
import jax
import jax.numpy as jnp
from jax.experimental import pallas as pl


def kernel(x, edge_index, W_l1, b_l1, W_r1, W_l2, b_l2, W_r2):
    raise NotImplementedError("write your pallas kernel here")


# speedup vs baseline: 5.5189x; 5.5189x over previous
import functools

import jax
import jax.numpy as jnp
from jax import lax
from jax.experimental import pallas as pl
from jax.experimental.pallas import tpu as pltpu
from jax.experimental.pallas import tpu_sc as plsc

N_NODES = 10000
N_EDGES = 320000
D = 128

NC, NS = 2, 16
NW = NC * NS
EPW = N_EDGES // NW
CHUNK = 80
NCHUNK = EPW // CHUNK
N_PAD = 10240
RPT = N_PAD // NS
SUB = RPT // CHUNK
DH = N_PAD // D


def _make_sc_agg():
    mesh = plsc.VectorSubcoreMesh(core_axis_name="c", subcore_axis_name="s")

    @functools.partial(
        pl.kernel,
        mesh=mesh,
        compiler_params=pltpu.CompilerParams(needs_layout_passes=False),
        out_type=[
            jax.ShapeDtypeStruct((NC, N_PAD, D), jnp.float32),
            jax.ShapeDtypeStruct((NW, N_PAD), jnp.float32),
        ],
        scratch_types=[
            pltpu.VMEM((CHUNK,), jnp.int32),
            pltpu.VMEM((CHUNK,), jnp.int32),
            pltpu.VMEM((CHUNK, D), jnp.float32),
            pltpu.VMEM((N_PAD,), jnp.float32),
            pltpu.VMEM_SHARED((N_PAD, D), jnp.float32),
            pltpu.SemaphoreType.DMA,
        ],
    )
    def agg(table, src, dst, zf, out_sum, out_deg,
            src_v, dst_v, rows_v, deg_v, acc, sem):
        c = lax.axis_index("c")
        s = lax.axis_index("s")
        wid = s * NC + c
        r0 = s * RPT

        pltpu.sync_copy(zf.at[pl.ds(0, CHUNK)], rows_v)

        def zero_piece(j, carry):
            pltpu.sync_copy(rows_v, acc.at[pl.ds(r0 + j * CHUNK, CHUNK)])
            return carry

        lax.fori_loop(0, SUB, zero_piece, 0)

        def zero_deg(i, carry):
            deg_v[pl.ds(i * 16, 16)] = jnp.zeros((16,), jnp.float32)
            return carry

        lax.fori_loop(0, N_PAD // 16, zero_deg, 0)

        plsc.subcore_barrier()

        base = wid * EPW

        def body(i, carry):
            off = base + i * CHUNK
            pltpu.sync_copy(src.at[pl.ds(off, CHUNK)], src_v)
            pltpu.sync_copy(dst.at[pl.ds(off, CHUNK)], dst_v)
            pltpu.async_copy(table.at[src_v], rows_v, sem).wait()
            pltpu.sync_copy(rows_v, acc.at[dst_v], add=True)
            for k in range(CHUNK // 16):
                d16 = dst_v[pl.ds(k * 16, 16)]
                plsc.addupdate_scatter(deg_v, [d16],
                                       jnp.ones((16,), jnp.float32))
            return carry

        lax.fori_loop(0, NCHUNK, body, 0)

        plsc.subcore_barrier()

        def out_piece(j, carry):
            rr = r0 + j * CHUNK
            pltpu.sync_copy(acc.at[pl.ds(rr, CHUNK)], rows_v)
            pltpu.sync_copy(rows_v, out_sum.at[c, pl.ds(rr, CHUNK)])
            return carry

        lax.fori_loop(0, SUB, out_piece, 0)
        pltpu.sync_copy(deg_v, out_deg.at[wid])

    return agg


_sc_agg = _make_sc_agg()


def _dense_body_relu(sum_ref, deg_ref, x_ref, wl_ref, bl_ref, wr_ref, out_ref):
    _dense_body(sum_ref, deg_ref, x_ref, wl_ref, bl_ref, wr_ref, out_ref, True)


def _dense_body_lin(sum_ref, deg_ref, x_ref, wl_ref, bl_ref, wr_ref, out_ref):
    _dense_body(sum_ref, deg_ref, x_ref, wl_ref, bl_ref, wr_ref, out_ref, False)


def _dense_body(sum_ref, deg_ref, x_ref, wl_ref, bl_ref, wr_ref, out_ref, relu):
    blk = out_ref.shape[0]
    s = sum_ref[0] + sum_ref[1]
    d = jnp.sum(deg_ref[...], axis=0)
    d = jnp.reshape(d, (blk, 1))
    agg = s * (1.0 / jnp.maximum(d, 1.0))
    h = jnp.dot(agg, wl_ref[...], preferred_element_type=jnp.float32)
    h = h + bl_ref[...]
    h = h + jnp.dot(x_ref[...], wr_ref[...], preferred_element_type=jnp.float32)
    if relu:
        h = jnp.maximum(h, 0.0)
    out_ref[...] = h


def _tc_dense(sum_parts, deg_parts, x, W_l, b_l, W_r, relu):
    BLK = 1024
    grid = N_PAD // BLK
    body = _dense_body_relu if relu else _dense_body_lin
    return pl.pallas_call(
        body,
        grid=(grid,),
        in_specs=[
            pl.BlockSpec((NC, BLK, D), lambda i: (0, i, 0)),
            pl.BlockSpec((NW, BLK), lambda i: (0, i)),
            pl.BlockSpec((BLK, D), lambda i: (i, 0)),
            pl.BlockSpec((D, D), lambda i: (0, 0)),
            pl.BlockSpec((1, D), lambda i: (0, 0)),
            pl.BlockSpec((D, D), lambda i: (0, 0)),
        ],
        out_specs=pl.BlockSpec((BLK, D), lambda i: (i, 0)),
        out_shape=jax.ShapeDtypeStruct((N_PAD, D), jnp.float32),
    )(sum_parts, deg_parts, x, W_l, b_l, W_r)


def kernel(x, edge_index, W_l1, b_l1, W_r1, W_l2, b_l2, W_r2):
    src = edge_index[0]
    dst = edge_index[1]
    x_pad = jnp.concatenate(
        [x, jnp.zeros((N_PAD - N_NODES, D), jnp.float32)], axis=0)
    zf = jnp.zeros((N_PAD, D), jnp.float32)
    sum1, deg = _sc_agg(x_pad, src, dst, zf)
    h = _tc_dense(sum1, deg, x_pad, W_l1, b_l1.reshape(1, D), W_r1, True)
    sum2, deg2 = _sc_agg(h, src, dst, zf)
    out = _tc_dense(sum2, deg2, h, W_l2, b_l2.reshape(1, D), W_r2, False)
    return out[:N_NODES]


# speedup vs baseline: 9.9253x; 1.7984x over previous
import functools

import jax
import jax.numpy as jnp
from jax import lax
from jax.experimental import pallas as pl
from jax.experimental.pallas import tpu as pltpu
from jax.experimental.pallas import tpu_sc as plsc

N_NODES = 10000
N_EDGES = 320000
D = 128

NC, NS = 2, 16
NW = NC * NS
EPW = N_EDGES // NW
CHUNK = 40
NCHUNK = EPW // CHUNK
NPAIR = NCHUNK // 2
N_PAD = 10240
RPT = N_PAD // NS
SUB = RPT // CHUNK


def _make_sc_agg(count_deg):
    mesh = plsc.VectorSubcoreMesh(core_axis_name="c", subcore_axis_name="s")

    out_type = [jax.ShapeDtypeStruct((NC, N_PAD, D), jnp.float32)]
    if count_deg:
        out_type.append(jax.ShapeDtypeStruct((NW, N_PAD), jnp.float32))

    @functools.partial(
        pl.kernel,
        mesh=mesh,
        compiler_params=pltpu.CompilerParams(needs_layout_passes=False),
        out_type=out_type,
        scratch_types=[
            pltpu.VMEM((EPW,), jnp.int32),
            pltpu.VMEM((EPW,), jnp.int32),
            pltpu.VMEM((CHUNK, D), jnp.float32),
            pltpu.VMEM((CHUNK, D), jnp.float32),
        ]
        + ([pltpu.VMEM((N_PAD,), jnp.float32)] if count_deg else [])
        + [
            pltpu.VMEM_SHARED((N_PAD, D), jnp.float32),
            pltpu.SemaphoreType.DMA,
            pltpu.SemaphoreType.DMA,
        ],
    )
    def agg(table, src, dst, zf, *refs):
        if count_deg:
            out_sum, out_deg = refs[0], refs[1]
            src_v, dst_v, rows0, rows1, deg_v, acc, sem0, sem1 = refs[2:]
        else:
            out_sum = refs[0]
            src_v, dst_v, rows0, rows1, acc, sem0, sem1 = refs[1:]
            deg_v = None
        rows = (rows0, rows1)
        sems = (sem0, sem1)

        c = lax.axis_index("c")
        s = lax.axis_index("s")
        wid = s * NC + c
        r0 = s * RPT
        base = wid * EPW

        pltpu.sync_copy(src.at[pl.ds(base, EPW)], src_v)
        pltpu.sync_copy(dst.at[pl.ds(base, EPW)], dst_v)

        pltpu.sync_copy(zf.at[pl.ds(0, CHUNK)], rows0)

        def zero_piece(j, carry):
            pltpu.sync_copy(rows0, acc.at[pl.ds(r0 + j * CHUNK, CHUNK)])
            return carry

        lax.fori_loop(0, SUB, zero_piece, 0)

        if count_deg:
            def zero_deg(i, carry):
                deg_v[pl.ds(i * 16, 16)] = jnp.zeros((16,), jnp.float32)
                return carry

            lax.fori_loop(0, N_PAD // 16, zero_deg, 0)

        plsc.subcore_barrier()

        def start_gather(chunk_i, b):
            off = chunk_i * CHUNK
            return pltpu.async_copy(
                table.at[src_v.at[pl.ds(off, CHUNK)]], rows[b], sems[b])

        def finish_chunk(chunk_i, b):
            off = chunk_i * CHUNK
            pltpu.make_async_copy(
                table.at[src_v.at[pl.ds(0, CHUNK)]], rows[b], sems[b]).wait()
            pltpu.sync_copy(rows[b], acc.at[dst_v.at[pl.ds(off, CHUNK)]],
                            add=True)

        start_gather(0, 0)
        start_gather(1, 1)

        def pair(j, carry):
            i0 = j * 2
            finish_chunk(i0, 0)

            @pl.when(i0 + 2 < NCHUNK)
            def _():
                start_gather(i0 + 2, 0)

            finish_chunk(i0 + 1, 1)

            @pl.when(i0 + 3 < NCHUNK)
            def _():
                start_gather(i0 + 3, 1)

            if count_deg:
                for k in range(2 * CHUNK // 16):
                    d16 = dst_v[pl.ds(i0 * CHUNK + k * 16, 16)]
                    plsc.addupdate_scatter(deg_v, [d16],
                                           jnp.ones((16,), jnp.float32))
            return carry

        lax.fori_loop(0, NPAIR, pair, 0)

        plsc.subcore_barrier()

        def out_piece(j, carry):
            rr = r0 + j * CHUNK
            pltpu.sync_copy(acc.at[pl.ds(rr, CHUNK)], rows0)
            pltpu.sync_copy(rows0, out_sum.at[c, pl.ds(rr, CHUNK)])
            return carry

        lax.fori_loop(0, SUB, out_piece, 0)
        if count_deg:
            pltpu.sync_copy(deg_v, out_deg.at[wid])

    return agg


_sc_agg_deg = _make_sc_agg(True)
_sc_agg = _make_sc_agg(False)


def _dense_body_relu(sum_ref, deg_ref, x_ref, wl_ref, bl_ref, wr_ref, out_ref):
    _dense_body(sum_ref, deg_ref, x_ref, wl_ref, bl_ref, wr_ref, out_ref, True)


def _dense_body_lin(sum_ref, deg_ref, x_ref, wl_ref, bl_ref, wr_ref, out_ref):
    _dense_body(sum_ref, deg_ref, x_ref, wl_ref, bl_ref, wr_ref, out_ref, False)


def _dense_body(sum_ref, deg_ref, x_ref, wl_ref, bl_ref, wr_ref, out_ref, relu):
    blk = out_ref.shape[0]
    s = sum_ref[0] + sum_ref[1]
    d = jnp.sum(deg_ref[...], axis=0)
    d = jnp.reshape(d, (blk, 1))
    agg = s * (1.0 / jnp.maximum(d, 1.0))
    h = jnp.dot(agg, wl_ref[...], preferred_element_type=jnp.float32)
    h = h + bl_ref[...]
    h = h + jnp.dot(x_ref[...], wr_ref[...], preferred_element_type=jnp.float32)
    if relu:
        h = jnp.maximum(h, 0.0)
    out_ref[...] = h


def _tc_dense(sum_parts, deg_parts, x, W_l, b_l, W_r, relu):
    BLK = 1024
    grid = N_PAD // BLK
    body = _dense_body_relu if relu else _dense_body_lin
    return pl.pallas_call(
        body,
        grid=(grid,),
        in_specs=[
            pl.BlockSpec((NC, BLK, D), lambda i: (0, i, 0)),
            pl.BlockSpec((NW, BLK), lambda i: (0, i)),
            pl.BlockSpec((BLK, D), lambda i: (i, 0)),
            pl.BlockSpec((D, D), lambda i: (0, 0)),
            pl.BlockSpec((1, D), lambda i: (0, 0)),
            pl.BlockSpec((D, D), lambda i: (0, 0)),
        ],
        out_specs=pl.BlockSpec((BLK, D), lambda i: (i, 0)),
        out_shape=jax.ShapeDtypeStruct((N_PAD, D), jnp.float32),
    )(sum_parts, deg_parts, x, W_l, b_l, W_r)


def kernel(x, edge_index, W_l1, b_l1, W_r1, W_l2, b_l2, W_r2):
    src = edge_index[0]
    dst = edge_index[1]
    x_pad = jnp.concatenate(
        [x, jnp.zeros((N_PAD - N_NODES, D), jnp.float32)], axis=0)
    zf = jnp.zeros((N_PAD, D), jnp.float32)
    sum1, deg = _sc_agg_deg(x_pad, src, dst, zf)
    h = _tc_dense(sum1, deg, x_pad, W_l1, b_l1.reshape(1, D), W_r1, True)
    (sum2,) = _sc_agg(h, src, dst, zf)
    out = _tc_dense(sum2, deg, h, W_l2, b_l2.reshape(1, D), W_r2, False)
    return out[:N_NODES]


# speedup vs baseline: 11.7035x; 1.1792x over previous
import functools

import jax
import jax.numpy as jnp
from jax import lax
from jax.experimental import pallas as pl
from jax.experimental.pallas import tpu as pltpu
from jax.experimental.pallas import tpu_sc as plsc

N_NODES = 10000
N_EDGES = 320000
D = 128

NC, NS = 2, 16
NW = NC * NS
EPW = N_EDGES // NW
CHUNK = 40
NCHUNK = EPW // CHUNK
N_PAD = 10240
RPT = N_PAD // NS
SUB = RPT // CHUNK


def _make_sc_agg(count_deg, nbuf):
    mesh = plsc.VectorSubcoreMesh(core_axis_name="c", subcore_axis_name="s")

    out_type = [jax.ShapeDtypeStruct((NC, N_PAD, D), jnp.float32)]
    if count_deg:
        out_type.append(jax.ShapeDtypeStruct((NW, N_PAD), jnp.float32))

    @functools.partial(
        pl.kernel,
        mesh=mesh,
        compiler_params=pltpu.CompilerParams(needs_layout_passes=False),
        out_type=out_type,
        scratch_types=[
            pltpu.VMEM((EPW,), jnp.int32),
            pltpu.VMEM((EPW,), jnp.int32),
        ]
        + [pltpu.VMEM((CHUNK, D), jnp.float32)] * nbuf
        + ([pltpu.VMEM((N_PAD,), jnp.float32)] if count_deg else [])
        + [pltpu.VMEM_SHARED((N_PAD, D), jnp.float32)]
        + [pltpu.SemaphoreType.DMA] * nbuf,
    )
    def agg(table, src, dst, zf, *refs):
        if count_deg:
            out_sum, out_deg = refs[0], refs[1]
            refs = refs[2:]
        else:
            out_sum = refs[0]
            refs = refs[1:]
        src_v, dst_v = refs[0], refs[1]
        rows = refs[2:2 + nbuf]
        if count_deg:
            deg_v = refs[2 + nbuf]
            acc = refs[3 + nbuf]
        else:
            deg_v = None
            acc = refs[2 + nbuf]
        sems = refs[-nbuf:]
        rows0 = rows[0]

        c = lax.axis_index("c")
        s = lax.axis_index("s")
        wid = s * NC + c
        r0 = s * RPT
        base = wid * EPW

        pltpu.sync_copy(src.at[pl.ds(base, EPW)], src_v)
        pltpu.sync_copy(dst.at[pl.ds(base, EPW)], dst_v)

        pltpu.sync_copy(zf.at[pl.ds(0, CHUNK)], rows0)

        def zero_piece(j, carry):
            pltpu.sync_copy(rows0, acc.at[pl.ds(r0 + j * CHUNK, CHUNK)])
            return carry

        lax.fori_loop(0, SUB, zero_piece, 0)

        if count_deg:
            def zero_deg(i, carry):
                deg_v[pl.ds(i * 16, 16)] = jnp.zeros((16,), jnp.float32)
                return carry

            lax.fori_loop(0, N_PAD // 16, zero_deg, 0)

        plsc.subcore_barrier()

        def start_gather(chunk_i, b):
            off = chunk_i * CHUNK
            return pltpu.async_copy(
                table.at[src_v.at[pl.ds(off, CHUNK)]], rows[b], sems[b])

        def finish_chunk(chunk_i, b):
            off = chunk_i * CHUNK
            pltpu.make_async_copy(
                table.at[src_v.at[pl.ds(0, CHUNK)]], rows[b], sems[b]).wait()
            pltpu.sync_copy(rows[b], acc.at[dst_v.at[pl.ds(off, CHUNK)]],
                            add=True)

        for b in range(nbuf):
            start_gather(b, b)

        ngrp = -(-NCHUNK // nbuf)

        def group(j, carry):
            i0 = j * nbuf
            for b in range(nbuf):
                i = i0 + b
                if NCHUNK % nbuf == 0:
                    finish_chunk(i, b)
                else:
                    @pl.when(i < NCHUNK)
                    def _(i=i, b=b):
                        finish_chunk(i, b)

                @pl.when(i + nbuf < NCHUNK)
                def _(i=i, b=b):
                    start_gather(i + nbuf, b)

            if count_deg:
                for k in range(nbuf * CHUNK // 16):
                    d16 = dst_v[pl.ds(i0 * CHUNK + k * 16, 16)]
                    plsc.addupdate_scatter(deg_v, [d16],
                                           jnp.ones((16,), jnp.float32))
            return carry

        lax.fori_loop(0, ngrp, group, 0)

        plsc.subcore_barrier()

        def out_piece(j, carry):
            rr = r0 + j * CHUNK
            pltpu.sync_copy(acc.at[pl.ds(rr, CHUNK)], rows0)
            pltpu.sync_copy(rows0, out_sum.at[c, pl.ds(rr, CHUNK)])
            return carry

        lax.fori_loop(0, SUB, out_piece, 0)
        if count_deg:
            pltpu.sync_copy(deg_v, out_deg.at[wid])

    return agg


_sc_agg_deg = _make_sc_agg(True, 2)
_sc_agg = _make_sc_agg(False, 4)


def _dense_body_relu(sum_ref, deg_ref, x_ref, wl_ref, bl_ref, wr_ref, out_ref):
    _dense_body(sum_ref, deg_ref, x_ref, wl_ref, bl_ref, wr_ref, out_ref, True)


def _dense_body_lin(sum_ref, deg_ref, x_ref, wl_ref, bl_ref, wr_ref, out_ref):
    _dense_body(sum_ref, deg_ref, x_ref, wl_ref, bl_ref, wr_ref, out_ref, False)


def _dense_body(sum_ref, deg_ref, x_ref, wl_ref, bl_ref, wr_ref, out_ref, relu):
    blk = out_ref.shape[0]
    s = sum_ref[0] + sum_ref[1]
    d = jnp.sum(deg_ref[...], axis=0)
    d = jnp.reshape(d, (blk, 1))
    agg = s * (1.0 / jnp.maximum(d, 1.0))
    h = jnp.dot(agg, wl_ref[...], preferred_element_type=jnp.float32)
    h = h + bl_ref[...]
    h = h + jnp.dot(x_ref[...], wr_ref[...], preferred_element_type=jnp.float32)
    if relu:
        h = jnp.maximum(h, 0.0)
    out_ref[...] = h


def _tc_dense(sum_parts, deg_parts, x, W_l, b_l, W_r, relu):
    BLK = 1024
    grid = N_PAD // BLK
    body = _dense_body_relu if relu else _dense_body_lin
    return pl.pallas_call(
        body,
        grid=(grid,),
        in_specs=[
            pl.BlockSpec((NC, BLK, D), lambda i: (0, i, 0)),
            pl.BlockSpec((NW, BLK), lambda i: (0, i)),
            pl.BlockSpec((BLK, D), lambda i: (i, 0)),
            pl.BlockSpec((D, D), lambda i: (0, 0)),
            pl.BlockSpec((1, D), lambda i: (0, 0)),
            pl.BlockSpec((D, D), lambda i: (0, 0)),
        ],
        out_specs=pl.BlockSpec((BLK, D), lambda i: (i, 0)),
        out_shape=jax.ShapeDtypeStruct((N_PAD, D), jnp.float32),
    )(sum_parts, deg_parts, x, W_l, b_l, W_r)


def kernel(x, edge_index, W_l1, b_l1, W_r1, W_l2, b_l2, W_r2):
    src = edge_index[0]
    dst = edge_index[1]
    x_pad = jnp.concatenate(
        [x, jnp.zeros((N_PAD - N_NODES, D), jnp.float32)], axis=0)
    zf = jnp.zeros((N_PAD, D), jnp.float32)
    sum1, deg = _sc_agg_deg(x_pad, src, dst, zf)
    h = _tc_dense(sum1, deg, x_pad, W_l1, b_l1.reshape(1, D), W_r1, True)
    (sum2,) = _sc_agg(h, src, dst, zf)
    out = _tc_dense(sum2, deg, h, W_l2, b_l2.reshape(1, D), W_r2, False)
    return out[:N_NODES]


# speedup vs baseline: 14.4339x; 1.2333x over previous
import functools

import jax
import jax.numpy as jnp
from jax import lax
from jax.experimental import pallas as pl
from jax.experimental.pallas import tpu as pltpu
from jax.experimental.pallas import tpu_sc as plsc

N_NODES = 10000
N_EDGES = 320000
D = 128

NC, NS = 2, 16
NW = NC * NS
EPW = N_EDGES // NW
CHUNK = 40
NCHUNK = EPW // CHUNK
N_PAD = 10240
RPT = N_PAD // NS
SUB = RPT // CHUNK


def _make_sc_agg(count_deg, nbuf, spans):
    mesh = plsc.VectorSubcoreMesh(core_axis_name="c", subcore_axis_name="s")
    ibuf = max(spans)

    out_type = [jax.ShapeDtypeStruct((NC, N_PAD, D), jnp.float32)]
    if count_deg:
        out_type.append(jax.ShapeDtypeStruct((NW, N_PAD), jnp.float32))

    @functools.partial(
        pl.kernel,
        mesh=mesh,
        compiler_params=pltpu.CompilerParams(needs_layout_passes=False),
        out_type=out_type,
        scratch_types=[
            pltpu.VMEM((ibuf,), jnp.int32),
            pltpu.VMEM((ibuf,), jnp.int32),
        ]
        + [pltpu.VMEM((CHUNK, D), jnp.float32)] * nbuf
        + ([pltpu.VMEM((N_PAD,), jnp.float32)] if count_deg else [])
        + [pltpu.VMEM_SHARED((N_PAD, D), jnp.float32)]
        + [pltpu.SemaphoreType.DMA] * nbuf,
    )
    def agg(table, src, dst, zf, *refs):
        if count_deg:
            out_sum, out_deg = refs[0], refs[1]
            refs = refs[2:]
        else:
            out_sum = refs[0]
            refs = refs[1:]
        src_v, dst_v = refs[0], refs[1]
        rows = refs[2:2 + nbuf]
        if count_deg:
            deg_v = refs[2 + nbuf]
            acc = refs[3 + nbuf]
        else:
            deg_v = None
            acc = refs[2 + nbuf]
        sems = refs[-nbuf:]
        rows0 = rows[0]

        c = lax.axis_index("c")
        s = lax.axis_index("s")
        wid = s * NC + c
        r0 = s * RPT
        base = wid * EPW

        pltpu.sync_copy(zf.at[pl.ds(0, CHUNK)], rows0)

        def zero_piece(j, carry):
            pltpu.sync_copy(rows0, acc.at[pl.ds(r0 + j * CHUNK, CHUNK)])
            return carry

        lax.fori_loop(0, SUB, zero_piece, 0)

        if count_deg:
            def zero_deg(i, carry):
                deg_v[pl.ds(i * 16, 16)] = jnp.zeros((16,), jnp.float32)
                return carry

            lax.fori_loop(0, N_PAD // 16, zero_deg, 0)

        plsc.subcore_barrier()

        def start_gather(chunk_i, b):
            off = chunk_i * CHUNK
            return pltpu.async_copy(
                table.at[src_v.at[pl.ds(off, CHUNK)]], rows[b], sems[b])

        def finish_chunk(chunk_i, b):
            off = chunk_i * CHUNK
            pltpu.make_async_copy(
                table.at[src_v.at[pl.ds(0, CHUNK)]], rows[b], sems[b]).wait()
            pltpu.sync_copy(rows[b], acc.at[dst_v.at[pl.ds(off, CHUNK)]],
                            add=True)

        def count_span_deg(e0, n16):
            for k in range(n16):
                d16 = dst_v[pl.ds(e0 + k * 16, 16)]
                plsc.addupdate_scatter(deg_v, [d16],
                                       jnp.ones((16,), jnp.float32))

        estart = 0
        for ecount in spans:
            pltpu.sync_copy(src.at[pl.ds(base + estart, ecount)],
                            src_v.at[pl.ds(0, ecount)])
            pltpu.sync_copy(dst.at[pl.ds(base + estart, ecount)],
                            dst_v.at[pl.ds(0, ecount)])
            nch = ecount // CHUNK
            ngrp = nch // nbuf
            ntail = nch % nbuf

            for b in range(nbuf):
                start_gather(b, b)

            def group(j, carry, nch=nch):
                i0 = j * nbuf
                for b in range(nbuf):
                    i = i0 + b
                    finish_chunk(i, b)

                    @pl.when(i + nbuf < nch)
                    def _(i=i, b=b):
                        start_gather(i + nbuf, b)

                if count_deg:
                    count_span_deg(i0 * CHUNK, nbuf * CHUNK // 16)
                return carry

            lax.fori_loop(0, ngrp, group, 0)

            for t in range(ntail):
                finish_chunk(ngrp * nbuf + t, t)
            if count_deg and ntail:
                count_span_deg(ngrp * nbuf * CHUNK, ntail * CHUNK // 16)

            estart += ecount

        plsc.subcore_barrier()

        def out_piece(j, carry):
            rr = r0 + j * CHUNK
            pltpu.sync_copy(acc.at[pl.ds(rr, CHUNK)], rows0)
            pltpu.sync_copy(rows0, out_sum.at[c, pl.ds(rr, CHUNK)])
            return carry

        lax.fori_loop(0, SUB, out_piece, 0)
        if count_deg:
            pltpu.sync_copy(deg_v, out_deg.at[wid])

    return agg


_sc_agg_deg = _make_sc_agg(True, 4, [5120, 4880])
_sc_agg = _make_sc_agg(False, 5, [EPW])


def _dense_body_relu(sum_ref, deg_ref, x_ref, wl_ref, bl_ref, wr_ref, out_ref):
    _dense_body(sum_ref, deg_ref, x_ref, wl_ref, bl_ref, wr_ref, out_ref, True)


def _dense_body_lin(sum_ref, deg_ref, x_ref, wl_ref, bl_ref, wr_ref, out_ref):
    _dense_body(sum_ref, deg_ref, x_ref, wl_ref, bl_ref, wr_ref, out_ref, False)


def _dense_body(sum_ref, deg_ref, x_ref, wl_ref, bl_ref, wr_ref, out_ref, relu):
    blk = out_ref.shape[0]
    s = sum_ref[0] + sum_ref[1]
    d = jnp.sum(deg_ref[...], axis=0)
    d = jnp.reshape(d, (blk, 1))
    agg = s * (1.0 / jnp.maximum(d, 1.0))
    h = jnp.dot(agg, wl_ref[...], preferred_element_type=jnp.float32)
    h = h + bl_ref[...]
    h = h + jnp.dot(x_ref[...], wr_ref[...], preferred_element_type=jnp.float32)
    if relu:
        h = jnp.maximum(h, 0.0)
    out_ref[...] = h


def _tc_dense(sum_parts, deg_parts, x, W_l, b_l, W_r, relu):
    BLK = 1024
    grid = N_PAD // BLK
    body = _dense_body_relu if relu else _dense_body_lin
    return pl.pallas_call(
        body,
        grid=(grid,),
        in_specs=[
            pl.BlockSpec((NC, BLK, D), lambda i: (0, i, 0)),
            pl.BlockSpec((NW, BLK), lambda i: (0, i)),
            pl.BlockSpec((BLK, D), lambda i: (i, 0)),
            pl.BlockSpec((D, D), lambda i: (0, 0)),
            pl.BlockSpec((1, D), lambda i: (0, 0)),
            pl.BlockSpec((D, D), lambda i: (0, 0)),
        ],
        out_specs=pl.BlockSpec((BLK, D), lambda i: (i, 0)),
        out_shape=jax.ShapeDtypeStruct((N_PAD, D), jnp.float32),
    )(sum_parts, deg_parts, x, W_l, b_l, W_r)


def kernel(x, edge_index, W_l1, b_l1, W_r1, W_l2, b_l2, W_r2):
    src = edge_index[0]
    dst = edge_index[1]
    x_pad = jnp.concatenate(
        [x, jnp.zeros((N_PAD - N_NODES, D), jnp.float32)], axis=0)
    zf = jnp.zeros((N_PAD, D), jnp.float32)
    sum1, deg = _sc_agg_deg(x_pad, src, dst, zf)
    h = _tc_dense(sum1, deg, x_pad, W_l1, b_l1.reshape(1, D), W_r1, True)
    (sum2,) = _sc_agg(h, src, dst, zf)
    out = _tc_dense(sum2, deg, h, W_l2, b_l2.reshape(1, D), W_r2, False)
    return out[:N_NODES]


# speedup vs baseline: 14.4359x; 1.0001x over previous
import functools

import jax
import jax.numpy as jnp
from jax import lax
from jax.experimental import pallas as pl
from jax.experimental.pallas import tpu as pltpu
from jax.experimental.pallas import tpu_sc as plsc

N_NODES = 10000
N_EDGES = 320000
D = 128

NC, NS = 2, 16
NW = NC * NS
EPW = N_EDGES // NW
CHUNK = 40
NCHUNK = EPW // CHUNK
N_PAD = 10240
RPT = N_PAD // NS
SUB = RPT // CHUNK


def _make_sc_agg(count_deg, nbuf, spans):
    mesh = plsc.VectorSubcoreMesh(core_axis_name="c", subcore_axis_name="s")
    ibuf = max(spans)

    out_type = [jax.ShapeDtypeStruct((NC, N_PAD, D), jnp.float32)]
    if count_deg:
        out_type.append(jax.ShapeDtypeStruct((NW, N_PAD), jnp.float32))

    @functools.partial(
        pl.kernel,
        mesh=mesh,
        compiler_params=pltpu.CompilerParams(needs_layout_passes=False),
        out_type=out_type,
        scratch_types=[
            pltpu.VMEM((ibuf,), jnp.int32),
            pltpu.VMEM((ibuf,), jnp.int32),
        ]
        + [pltpu.VMEM((CHUNK, D), jnp.float32)] * nbuf
        + ([pltpu.VMEM((N_PAD,), jnp.float32)] if count_deg else [])
        + [pltpu.VMEM_SHARED((N_PAD, D), jnp.float32)]
        + [pltpu.SemaphoreType.DMA] * nbuf,
    )
    def agg(table, src, dst, zf, *refs):
        if count_deg:
            out_sum, out_deg = refs[0], refs[1]
            refs = refs[2:]
        else:
            out_sum = refs[0]
            refs = refs[1:]
        src_v, dst_v = refs[0], refs[1]
        rows = refs[2:2 + nbuf]
        if count_deg:
            deg_v = refs[2 + nbuf]
            acc = refs[3 + nbuf]
        else:
            deg_v = None
            acc = refs[2 + nbuf]
        sems = refs[-nbuf:]
        rows0 = rows[0]

        c = lax.axis_index("c")
        s = lax.axis_index("s")
        wid = s * NC + c
        r0 = s * RPT
        base = wid * EPW

        pltpu.sync_copy(zf.at[pl.ds(0, CHUNK)], rows0)

        def zero_piece(j, carry):
            pltpu.sync_copy(rows0, acc.at[pl.ds(r0 + j * CHUNK, CHUNK)])
            return carry

        lax.fori_loop(0, SUB, zero_piece, 0)

        if count_deg:
            def zero_deg(i, carry):
                deg_v[pl.ds(i * 16, 16)] = jnp.zeros((16,), jnp.float32)
                return carry

            lax.fori_loop(0, N_PAD // 16, zero_deg, 0)

        plsc.subcore_barrier()

        def start_gather(chunk_i, b):
            off = chunk_i * CHUNK
            return pltpu.async_copy(
                table.at[src_v.at[pl.ds(off, CHUNK)]], rows[b], sems[b])

        def finish_chunk(chunk_i, b):
            off = chunk_i * CHUNK
            pltpu.make_async_copy(
                table.at[src_v.at[pl.ds(0, CHUNK)]], rows[b], sems[b]).wait()
            pltpu.sync_copy(rows[b], acc.at[dst_v.at[pl.ds(off, CHUNK)]],
                            add=True)

        def count_span_deg(e0, n16):
            for k in range(n16):
                d16 = dst_v[pl.ds(e0 + k * 16, 16)]
                plsc.addupdate_scatter(deg_v, [d16],
                                       jnp.ones((16,), jnp.float32))

        estart = 0
        for ecount in spans:
            pltpu.sync_copy(src.at[pl.ds(base + estart, ecount)],
                            src_v.at[pl.ds(0, ecount)])
            pltpu.sync_copy(dst.at[pl.ds(base + estart, ecount)],
                            dst_v.at[pl.ds(0, ecount)])
            nch = ecount // CHUNK
            ngrp = nch // nbuf
            ntail = nch % nbuf

            for b in range(nbuf):
                start_gather(b, b)

            def group(j, carry, nch=nch):
                i0 = j * nbuf
                for b in range(nbuf):
                    i = i0 + b
                    finish_chunk(i, b)

                    @pl.when(i + nbuf < nch)
                    def _(i=i, b=b):
                        start_gather(i + nbuf, b)

                if count_deg:
                    count_span_deg(i0 * CHUNK, nbuf * CHUNK // 16)
                return carry

            lax.fori_loop(0, ngrp, group, 0)

            for t in range(ntail):
                finish_chunk(ngrp * nbuf + t, t)
            if count_deg and ntail:
                count_span_deg(ngrp * nbuf * CHUNK, ntail * CHUNK // 16)

            estart += ecount

        plsc.subcore_barrier()

        def out_piece(j, carry):
            rr = r0 + j * CHUNK
            pltpu.sync_copy(acc.at[pl.ds(rr, CHUNK)], rows0)
            pltpu.sync_copy(rows0, out_sum.at[c, pl.ds(rr, CHUNK)])
            return carry

        lax.fori_loop(0, SUB, out_piece, 0)
        if count_deg:
            pltpu.sync_copy(deg_v, out_deg.at[wid])

    return agg


_sc_agg_deg = _make_sc_agg(True, 4, [5120, 4880])
_sc_agg = _make_sc_agg(False, 5, [EPW])


BLK = 1024


def _pre_body(x_ref, wr_ref, out_ref):
    out_ref[...] = jnp.dot(x_ref[...], wr_ref[...],
                           preferred_element_type=jnp.float32)


def _tc_pre(x, W_r):
    return pl.pallas_call(
        _pre_body,
        grid=(N_PAD // BLK,),
        in_specs=[
            pl.BlockSpec((BLK, D), lambda i: (i, 0)),
            pl.BlockSpec((D, D), lambda i: (0, 0)),
        ],
        out_specs=pl.BlockSpec((BLK, D), lambda i: (i, 0)),
        out_shape=jax.ShapeDtypeStruct((N_PAD, D), jnp.float32),
    )(x, W_r)


def _post_body_relu(sum_ref, deg_ref, xr_ref, wl_ref, bl_ref, out_ref):
    _post_body(sum_ref, deg_ref, xr_ref, wl_ref, bl_ref, out_ref, True)


def _post_body_lin(sum_ref, deg_ref, xr_ref, wl_ref, bl_ref, out_ref):
    _post_body(sum_ref, deg_ref, xr_ref, wl_ref, bl_ref, out_ref, False)


def _post_body(sum_ref, deg_ref, xr_ref, wl_ref, bl_ref, out_ref, relu):
    blk = out_ref.shape[0]
    s = sum_ref[0] + sum_ref[1]
    d = jnp.sum(deg_ref[...], axis=0)
    d = jnp.reshape(d, (blk, 1))
    agg = s * (1.0 / jnp.maximum(d, 1.0))
    h = jnp.dot(agg, wl_ref[...], preferred_element_type=jnp.float32)
    h = h + bl_ref[...] + xr_ref[...]
    if relu:
        h = jnp.maximum(h, 0.0)
    out_ref[...] = h


def _tc_post(sum_parts, deg_parts, xr, W_l, b_l, relu):
    body = _post_body_relu if relu else _post_body_lin
    return pl.pallas_call(
        body,
        grid=(N_PAD // BLK,),
        in_specs=[
            pl.BlockSpec((NC, BLK, D), lambda i: (0, i, 0)),
            pl.BlockSpec((NW, BLK), lambda i: (0, i)),
            pl.BlockSpec((BLK, D), lambda i: (i, 0)),
            pl.BlockSpec((D, D), lambda i: (0, 0)),
            pl.BlockSpec((1, D), lambda i: (0, 0)),
        ],
        out_specs=pl.BlockSpec((BLK, D), lambda i: (i, 0)),
        out_shape=jax.ShapeDtypeStruct((N_PAD, D), jnp.float32),
    )(sum_parts, deg_parts, xr, W_l, b_l)


def kernel(x, edge_index, W_l1, b_l1, W_r1, W_l2, b_l2, W_r2):
    src = edge_index[0]
    dst = edge_index[1]
    x_pad = jnp.concatenate(
        [x, jnp.zeros((N_PAD - N_NODES, D), jnp.float32)], axis=0)
    zf = jnp.zeros((CHUNK, D), jnp.float32)
    xr1 = _tc_pre(x_pad, W_r1)
    sum1, deg = _sc_agg_deg(x_pad, src, dst, zf)
    h = _tc_post(sum1, deg, xr1, W_l1, b_l1.reshape(1, D), True)
    xr2 = _tc_pre(h, W_r2)
    (sum2,) = _sc_agg(h, src, dst, zf)
    out = _tc_post(sum2, deg, xr2, W_l2, b_l2.reshape(1, D), False)
    return out[:N_NODES]
